# baseline (speedup 1.0000x reference)
import functools

import jax
import jax.numpy as jnp
from jax import lax
from jax.experimental import pallas as pl
from jax.experimental.pallas import tpu as pltpu
from jax.experimental.pallas import tpu_sc as plsc

USER_N = 10000
ITEM_N = 40000
N_NODES = USER_N + ITEM_N
D = 64
HD = D // 2
E = 800000
LMBD = 0.0001
BATCH = 4096

NC = 2
NS = 16
NPAD = 50176
RPT = NPAD // NS
EPT = E // NS
K = 400
NCHUNK = EPT // K
BW = BATCH // (NC * NS)
NROWBLK = 512
EPS = 1e-12

_f32 = jnp.float32
_i32 = jnp.int32


def _iota16():
    return lax.iota(_i32, 16)


def _hsum16(v):
    ii = _iota16()
    for k in (8, 4, 2, 1):
        v = v + v[ii ^ k]
    return v


def _accum_body(edges_hbm, h_hbm, out_hbm,
                idx0, idx1, rows0, rows1,
                acc_sh, si0, si1, sg0, sg1):
    c = lax.axis_index("c")
    s = lax.axis_index("s")

    z = jnp.zeros((16,), _f32)

    def _zrow(r, _):
        for j in range(HD // 16):
            rows0[r, pl.ds(j * 16, 16)] = z
        return 0

    lax.fori_loop(0, K, _zrow, 0)
    nz = RPT // K
    rem = RPT - nz * K
    for r in range(nz):
        pltpu.async_copy(rows0, acc_sh.at[pl.ds(s * RPT + r * K, K)], si0)
    if rem:
        pltpu.async_copy(rows0.at[pl.ds(0, rem)],
                         acc_sh.at[pl.ds(s * RPT + nz * K, rem)], si0)
    for r in range(nz):
        pltpu.make_async_copy(
            rows0, acc_sh.at[pl.ds(s * RPT + r * K, K)], si0).wait()
    if rem:
        pltpu.make_async_copy(
            rows0.at[pl.ds(0, rem)],
            acc_sh.at[pl.ds(s * RPT + nz * K, rem)], si0).wait()
    plsc.subcore_barrier()

    sets = ((idx0, rows0, si0, sg0), (idx1, rows1, si1, sg1))

    def _start_idx(i, st):
        ev, _, si, _ = st
        pltpu.async_copy(edges_hbm.at[s, i], ev, si)

    def _wait_idx(st):
        ev, _, si, _ = st
        pltpu.make_async_copy(edges_hbm.at[s, 0], ev, si).wait()

    def _start_gather(st):
        ev, rv, _, sg = st
        pltpu.async_copy(h_hbm.at[c].at[ev.at[0]], rv, sg)

    def _wait_gather(st):
        ev, rv, _, sg = st
        pltpu.make_async_copy(h_hbm.at[c].at[ev.at[0]], rv, sg).wait()

    def _process(st):
        ev, rv, _, _ = st

        def _scale(g, _):
            wgrp = lax.bitcast_convert_type(ev[2, pl.ds(g * 16, 16)], _f32)
            for m in range(16):
                e = g * 16 + m
                we = wgrp[m]
                for j in range(HD // 16):
                    sl = pl.ds(j * 16, 16)
                    rv[e, sl] = rv[e, sl] * we
            return 0

        lax.fori_loop(0, K // 16, _scale, 0)
        pltpu.sync_copy(rv, acc_sh.at[ev.at[1]], add=True)

    _start_idx(0, sets[0])
    _wait_idx(sets[0])
    _start_gather(sets[0])
    _start_idx(1, sets[1])

    def _pair(p, _):
        _wait_idx(sets[1])
        _wait_gather(sets[0])
        _start_gather(sets[1])
        _process(sets[0])
        _start_idx(2 * p + 2, sets[0])
        _wait_idx(sets[0])
        _wait_gather(sets[1])
        _start_gather(sets[0])
        _process(sets[1])

        @pl.when(p < (NCHUNK - 1) // 2 - 1)
        def _():
            _start_idx(2 * p + 3, sets[1])

        return 0

    lax.fori_loop(0, (NCHUNK - 1) // 2, _pair, 0)
    _wait_gather(sets[0])
    _process(sets[0])
    plsc.subcore_barrier()

    sl = pl.ds(s * RPT, RPT)
    pltpu.sync_copy(acc_sh.at[sl], out_hbm.at[c].at[sl])


def _make_accum():
    mesh = plsc.VectorSubcoreMesh(core_axis_name="c", subcore_axis_name="s")
    return pl.kernel(
        _accum_body,
        out_type=jax.ShapeDtypeStruct((NC, NPAD, HD), _f32),
        mesh=mesh,
        compiler_params=pltpu.CompilerParams(use_tc_tiling_on_sc=False),
        scratch_types=[
            pltpu.VMEM((3, K), _i32),
            pltpu.VMEM((3, K), _i32),
            pltpu.VMEM((K, HD), _f32),
            pltpu.VMEM((K, HD), _f32),
            pltpu.VMEM_SHARED((NPAD, HD), _f32),
            pltpu.SemaphoreType.DMA,
            pltpu.SemaphoreType.DMA,
            pltpu.SemaphoreType.DMA,
            pltpu.SemaphoreType.DMA,
        ],
    )


def _norm_body(agg_ref, out_ref):
    x = agg_ref[...]
    ss = jnp.sum(x * x, axis=(0, 2))
    inv = 1.0 / jnp.maximum(jnp.sqrt(ss), EPS)
    out_ref[...] = x * inv[None, :, None]


def _normalize(agg):
    return pl.pallas_call(
        _norm_body,
        grid=(NPAD // NROWBLK,),
        in_specs=[pl.BlockSpec((NC, NROWBLK, HD), lambda i: (0, i, 0))],
        out_specs=pl.BlockSpec((NC, NROWBLK, HD), lambda i: (0, i, 0)),
        out_shape=jax.ShapeDtypeStruct((NC, NPAD, HD), _f32),
    )(agg)


def _final_body(h0, h1, h2, h3, uid_hbm, pid_hbm, nid_hbm,
                pos_hbm, neg_hbm, reg_hbm,
                idx_v, ua_v, ub_v, pa_v, pb_v, na_v, nb_v,
                pos_v, neg_v, reg_v, sem):
    c = lax.axis_index("c")
    s = lax.axis_index("s")
    wid = s * NC + c
    base = wid * BW
    ii = _iota16()

    def _sumrows(id_hbm, bufa, bufb):
        pltpu.sync_copy(id_hbm.at[pl.ds(base, BW)], idx_v)
        pltpu.async_copy(h0.at[0].at[idx_v], bufa, sem).wait()
        pltpu.async_copy(h0.at[1].at[idx_v], bufb, sem).wait()
        for t in (h1, h2, h3):
            pltpu.async_copy(t.at[0].at[idx_v], bufa, sem, add=True).wait()
            pltpu.async_copy(t.at[1].at[idx_v], bufb, sem, add=True).wait()

    _sumrows(uid_hbm, ua_v, ub_v)
    _sumrows(pid_hbm, pa_v, pb_v)
    _sumrows(nid_hbm, na_v, nb_v)

    racc0 = jnp.zeros((16,), _f32)

    def _grp(g, racc):
        pos16 = jnp.zeros((16,), _f32)
        neg16 = jnp.zeros((16,), _f32)
        for m in range(16):
            e = g * 16 + m
            pacc = jnp.zeros((16,), _f32)
            nacc = jnp.zeros((16,), _f32)
            for (ub, pb, nb) in ((ua_v, pa_v, na_v), (ub_v, pb_v, nb_v)):
                for j in range(HD // 16):
                    sl = pl.ds(j * 16, 16)
                    gu = ub[e, sl]
                    gp = pb[e, sl]
                    gn = nb[e, sl]
                    pacc = pacc + gu * gp
                    nacc = nacc + gu * gn
                    racc = racc + gu * gu + gp * gp + gn * gn
            lane = ii == m
            pos16 = jnp.where(lane, _hsum16(pacc), pos16)
            neg16 = jnp.where(lane, _hsum16(nacc), neg16)
        pos_v[pl.ds(g * 16, 16)] = pos16 * 0.0625
        neg_v[pl.ds(g * 16, 16)] = neg16 * 0.0625
        return racc

    racc = lax.fori_loop(0, BW // 16, _grp, racc0)
    reg_v[pl.ds(0, 16)] = racc * 0.0625

    pltpu.sync_copy(pos_v, pos_hbm.at[pl.ds(base, BW)])
    pltpu.sync_copy(neg_v, neg_hbm.at[pl.ds(base, BW)])
    pltpu.sync_copy(reg_v, reg_hbm.at[wid])


def _make_final():
    mesh = plsc.VectorSubcoreMesh(core_axis_name="c", subcore_axis_name="s")
    return pl.kernel(
        _final_body,
        out_type=(
            jax.ShapeDtypeStruct((BATCH,), _f32),
            jax.ShapeDtypeStruct((BATCH,), _f32),
            jax.ShapeDtypeStruct((NC * NS, 16), _f32),
        ),
        mesh=mesh,
        compiler_params=pltpu.CompilerParams(use_tc_tiling_on_sc=False),
        scratch_types=[
            pltpu.VMEM((BW,), _i32),
            pltpu.VMEM((BW, HD), _f32),
            pltpu.VMEM((BW, HD), _f32),
            pltpu.VMEM((BW, HD), _f32),
            pltpu.VMEM((BW, HD), _f32),
            pltpu.VMEM((BW, HD), _f32),
            pltpu.VMEM((BW, HD), _f32),
            pltpu.VMEM((BW,), _f32),
            pltpu.VMEM((BW,), _f32),
            pltpu.VMEM((16,), _f32),
            pltpu.SemaphoreType.DMA,
        ],
    )


def _loss_body(pos_ref, neg_ref, reg_ref, out_ref):
    z = pos_ref[...] - neg_ref[...]
    loss = -jnp.mean(jnp.log(jax.nn.sigmoid(z)))
    reg = jnp.sum(reg_ref[...])
    out_ref[...] = jnp.full((1, 1), loss + LMBD * (reg * 0.5) / BATCH, _f32)


def kernel(edge_index, edge_weight, user_table, item_table,
           user_id, item_id, neg_item_id):
    src = edge_index[0].astype(_i32)
    dst = edge_index[1].astype(_i32)
    wbits = lax.bitcast_convert_type(edge_weight.astype(_f32), _i32)
    edges = (jnp.stack([src, dst, wbits])
             .reshape(3, NS, NCHUNK, K)
             .transpose(1, 2, 0, 3))
    pad = jnp.zeros((NPAD - N_NODES, D), _f32)
    h0full = jnp.concatenate([user_table, item_table, pad], axis=0)
    h0 = jnp.stack([h0full[:, :HD], h0full[:, HD:]])

    accum = _make_accum()
    h1 = _normalize(accum(edges, h0))
    h2 = _normalize(accum(edges, h1))
    h3 = _normalize(accum(edges, h2))

    final = _make_final()
    pos, neg, regp = final(
        h0, h1, h2, h3,
        user_id.astype(_i32),
        (item_id + USER_N).astype(_i32),
        (neg_item_id + USER_N).astype(_i32),
    )

    out = pl.pallas_call(
        _loss_body,
        out_shape=jax.ShapeDtypeStruct((1, 1), _f32),
    )(pos.reshape(32, 128), neg.reshape(32, 128), regp)
    return out[0, 0]


# speedup vs baseline: 1.0214x; 1.0214x over previous
import functools

import jax
import jax.numpy as jnp
from jax import lax
from jax.experimental import pallas as pl
from jax.experimental.pallas import tpu as pltpu
from jax.experimental.pallas import tpu_sc as plsc

USER_N = 10000
ITEM_N = 40000
N_NODES = USER_N + ITEM_N
D = 64
HD = D // 2
E = 800000
LMBD = 0.0001
BATCH = 4096

NC = 2
NS = 16
NPAD = 50176
RPT = NPAD // NS
EPT = E // NS
K = 400
NCHUNK = EPT // K
BW = BATCH // (NC * NS)
NROWBLK = 512
EPS = 1e-12

_f32 = jnp.float32
_i32 = jnp.int32


def _iota16():
    return lax.iota(_i32, 16)


def _hsum16(v):
    ii = _iota16()
    for k in (8, 4, 2, 1):
        v = v + v[ii ^ k]
    return v


def _accum_body(src_hbm, dst_hbm, w_hbm, h_hbm, out_hbm,
                idx0, idx1, rows0, rows1,
                acc_sh, si0, si1, sg0, sg1):
    c = lax.axis_index("c")
    s = lax.axis_index("s")

    z = jnp.zeros((16,), _f32)

    def _zrow(r, _):
        for j in range(HD // 16):
            rows0[r, pl.ds(j * 16, 16)] = z
        return 0

    lax.fori_loop(0, K, _zrow, 0)
    nz = RPT // K
    rem = RPT - nz * K
    for r in range(nz):
        pltpu.async_copy(rows0, acc_sh.at[pl.ds(s * RPT + r * K, K)], si0)
    if rem:
        pltpu.async_copy(rows0.at[pl.ds(0, rem)],
                         acc_sh.at[pl.ds(s * RPT + nz * K, rem)], si0)
    for r in range(nz):
        pltpu.make_async_copy(
            rows0, acc_sh.at[pl.ds(s * RPT + r * K, K)], si0).wait()
    if rem:
        pltpu.make_async_copy(
            rows0.at[pl.ds(0, rem)],
            acc_sh.at[pl.ds(s * RPT + nz * K, rem)], si0).wait()
    plsc.subcore_barrier()

    sets = ((idx0, rows0, si0, sg0), (idx1, rows1, si1, sg1))

    def _start_idx(i, st):
        ev, _, si, _ = st
        base = s * EPT + i * K
        pltpu.async_copy(src_hbm.at[pl.ds(base, K)], ev.at[0], si)
        pltpu.async_copy(dst_hbm.at[pl.ds(base, K)], ev.at[1], si)
        pltpu.async_copy(w_hbm.at[pl.ds(base, K)], ev.at[2], si)

    def _wait_idx(st):
        ev, _, si, _ = st
        pltpu.make_async_copy(src_hbm.at[pl.ds(0, K)], ev.at[0], si).wait()
        pltpu.make_async_copy(dst_hbm.at[pl.ds(0, K)], ev.at[1], si).wait()
        pltpu.make_async_copy(w_hbm.at[pl.ds(0, K)], ev.at[2], si).wait()

    def _start_gather(st):
        ev, rv, _, sg = st
        pltpu.async_copy(h_hbm.at[c].at[ev.at[0]], rv, sg)

    def _wait_gather(st):
        ev, rv, _, sg = st
        pltpu.make_async_copy(h_hbm.at[c].at[ev.at[0]], rv, sg).wait()

    def _process(st):
        ev, rv, _, _ = st

        def _scale(g, _):
            wgrp = lax.bitcast_convert_type(ev[2, pl.ds(g * 16, 16)], _f32)
            for m in range(16):
                e = g * 16 + m
                we = wgrp[m]
                for j in range(HD // 16):
                    sl = pl.ds(j * 16, 16)
                    rv[e, sl] = rv[e, sl] * we
            return 0

        lax.fori_loop(0, K // 16, _scale, 0)
        pltpu.sync_copy(rv, acc_sh.at[ev.at[1]], add=True)

    _start_idx(0, sets[0])
    _wait_idx(sets[0])
    _start_gather(sets[0])
    _start_idx(1, sets[1])

    def _pair(p, _):
        _wait_idx(sets[1])
        _wait_gather(sets[0])
        _start_gather(sets[1])
        _process(sets[0])
        _start_idx(2 * p + 2, sets[0])
        _wait_idx(sets[0])
        _wait_gather(sets[1])
        _start_gather(sets[0])
        _process(sets[1])

        @pl.when(p < (NCHUNK - 1) // 2 - 1)
        def _():
            _start_idx(2 * p + 3, sets[1])

        return 0

    lax.fori_loop(0, (NCHUNK - 1) // 2, _pair, 0)
    _wait_gather(sets[0])
    _process(sets[0])
    plsc.subcore_barrier()

    sl = pl.ds(s * RPT, RPT)
    pltpu.sync_copy(acc_sh.at[sl], out_hbm.at[c].at[sl])


def _make_accum():
    mesh = plsc.VectorSubcoreMesh(core_axis_name="c", subcore_axis_name="s")
    return pl.kernel(
        _accum_body,
        out_type=jax.ShapeDtypeStruct((NC, NPAD, HD), _f32),
        mesh=mesh,
        compiler_params=pltpu.CompilerParams(use_tc_tiling_on_sc=False),
        scratch_types=[
            pltpu.VMEM((3, K), _i32),
            pltpu.VMEM((3, K), _i32),
            pltpu.VMEM((K, HD), _f32),
            pltpu.VMEM((K, HD), _f32),
            pltpu.VMEM_SHARED((NPAD, HD), _f32),
            pltpu.SemaphoreType.DMA,
            pltpu.SemaphoreType.DMA,
            pltpu.SemaphoreType.DMA,
            pltpu.SemaphoreType.DMA,
        ],
    )


def _norm_body(agg_ref, out_ref):
    x = agg_ref[...]
    ss = jnp.sum(x * x, axis=(0, 2))
    inv = 1.0 / jnp.maximum(jnp.sqrt(ss), EPS)
    out_ref[...] = x * inv[None, :, None]


def _normalize(agg):
    return pl.pallas_call(
        _norm_body,
        grid=(NPAD // NROWBLK,),
        in_specs=[pl.BlockSpec((NC, NROWBLK, HD), lambda i: (0, i, 0))],
        out_specs=pl.BlockSpec((NC, NROWBLK, HD), lambda i: (0, i, 0)),
        out_shape=jax.ShapeDtypeStruct((NC, NPAD, HD), _f32),
    )(agg)


def _final_body(h0, h1, h2, h3, uid_hbm, pid_hbm, nid_hbm,
                pos_hbm, neg_hbm, reg_hbm,
                idx_v, ua_v, ub_v, pa_v, pb_v, na_v, nb_v,
                pos_v, neg_v, reg_v, sem):
    c = lax.axis_index("c")
    s = lax.axis_index("s")
    wid = s * NC + c
    base = wid * BW
    ii = _iota16()

    def _sumrows(id_hbm, bufa, bufb):
        pltpu.sync_copy(id_hbm.at[pl.ds(base, BW)], idx_v)
        pltpu.async_copy(h0.at[0].at[idx_v], bufa, sem).wait()
        pltpu.async_copy(h0.at[1].at[idx_v], bufb, sem).wait()
        for t in (h1, h2, h3):
            pltpu.async_copy(t.at[0].at[idx_v], bufa, sem, add=True).wait()
            pltpu.async_copy(t.at[1].at[idx_v], bufb, sem, add=True).wait()

    _sumrows(uid_hbm, ua_v, ub_v)
    _sumrows(pid_hbm, pa_v, pb_v)
    _sumrows(nid_hbm, na_v, nb_v)

    racc0 = jnp.zeros((16,), _f32)

    def _grp(g, racc):
        pos16 = jnp.zeros((16,), _f32)
        neg16 = jnp.zeros((16,), _f32)
        for m in range(16):
            e = g * 16 + m
            pacc = jnp.zeros((16,), _f32)
            nacc = jnp.zeros((16,), _f32)
            for (ub, pb, nb) in ((ua_v, pa_v, na_v), (ub_v, pb_v, nb_v)):
                for j in range(HD // 16):
                    sl = pl.ds(j * 16, 16)
                    gu = ub[e, sl]
                    gp = pb[e, sl]
                    gn = nb[e, sl]
                    pacc = pacc + gu * gp
                    nacc = nacc + gu * gn
                    racc = racc + gu * gu + gp * gp + gn * gn
            lane = ii == m
            pos16 = jnp.where(lane, _hsum16(pacc), pos16)
            neg16 = jnp.where(lane, _hsum16(nacc), neg16)
        pos_v[pl.ds(g * 16, 16)] = pos16 * 0.0625
        neg_v[pl.ds(g * 16, 16)] = neg16 * 0.0625
        return racc

    racc = lax.fori_loop(0, BW // 16, _grp, racc0)
    reg_v[pl.ds(0, 16)] = racc * 0.0625

    pltpu.sync_copy(pos_v, pos_hbm.at[pl.ds(base, BW)])
    pltpu.sync_copy(neg_v, neg_hbm.at[pl.ds(base, BW)])
    pltpu.sync_copy(reg_v, reg_hbm.at[wid])


def _make_final():
    mesh = plsc.VectorSubcoreMesh(core_axis_name="c", subcore_axis_name="s")
    return pl.kernel(
        _final_body,
        out_type=(
            jax.ShapeDtypeStruct((BATCH,), _f32),
            jax.ShapeDtypeStruct((BATCH,), _f32),
            jax.ShapeDtypeStruct((NC * NS, 16), _f32),
        ),
        mesh=mesh,
        compiler_params=pltpu.CompilerParams(use_tc_tiling_on_sc=False),
        scratch_types=[
            pltpu.VMEM((BW,), _i32),
            pltpu.VMEM((BW, HD), _f32),
            pltpu.VMEM((BW, HD), _f32),
            pltpu.VMEM((BW, HD), _f32),
            pltpu.VMEM((BW, HD), _f32),
            pltpu.VMEM((BW, HD), _f32),
            pltpu.VMEM((BW, HD), _f32),
            pltpu.VMEM((BW,), _f32),
            pltpu.VMEM((BW,), _f32),
            pltpu.VMEM((16,), _f32),
            pltpu.SemaphoreType.DMA,
        ],
    )


def _loss_body(pos_ref, neg_ref, reg_ref, out_ref):
    z = pos_ref[...] - neg_ref[...]
    loss = -jnp.mean(jnp.log(jax.nn.sigmoid(z)))
    reg = jnp.sum(reg_ref[...])
    out_ref[...] = jnp.full((1, 1), loss + LMBD * (reg * 0.5) / BATCH, _f32)


def kernel(edge_index, edge_weight, user_table, item_table,
           user_id, item_id, neg_item_id):
    src = edge_index[0].astype(_i32)
    dst = edge_index[1].astype(_i32)
    wbits = lax.bitcast_convert_type(edge_weight.astype(_f32), _i32)
    pad = jnp.zeros((NPAD - N_NODES, D), _f32)
    h0full = jnp.concatenate([user_table, item_table, pad], axis=0)
    h0 = jnp.stack([h0full[:, :HD], h0full[:, HD:]])

    accum = _make_accum()
    h1 = _normalize(accum(src, dst, wbits, h0))
    h2 = _normalize(accum(src, dst, wbits, h1))
    h3 = _normalize(accum(src, dst, wbits, h2))

    final = _make_final()
    pos, neg, regp = final(
        h0, h1, h2, h3,
        user_id.astype(_i32),
        (item_id + USER_N).astype(_i32),
        (neg_item_id + USER_N).astype(_i32),
    )

    out = pl.pallas_call(
        _loss_body,
        out_shape=jax.ShapeDtypeStruct((1, 1), _f32),
    )(pos.reshape(32, 128), neg.reshape(32, 128), regp)
    return out[0, 0]


# speedup vs baseline: 1.4410x; 1.4108x over previous
import functools

import jax
import jax.numpy as jnp
from jax import lax
from jax.experimental import pallas as pl
from jax.experimental.pallas import tpu as pltpu
from jax.experimental.pallas import tpu_sc as plsc

USER_N = 10000
ITEM_N = 40000
N_NODES = USER_N + ITEM_N
D = 64
HD = D // 2
E = 800000
LMBD = 0.0001
BATCH = 4096

NC = 2
NS = 16
NPAD = 50176
RPT = NPAD // NS
EPT = E // NS
K = 400
NCHUNK = EPT // K
BW = BATCH // (NC * NS)
NROWBLK = 896
EPS = 1e-12

_f32 = jnp.float32
_i32 = jnp.int32


def _iota16():
    return lax.iota(_i32, 16)


def _hsum16(v):
    ii = _iota16()
    for k in (8, 4, 2, 1):
        v = v + v[ii ^ k]
    return v


def _accum_body(src_hbm, dst_hbm, w_hbm, h_hbm, out_hbm,
                idx0, idx1, rows0, rows1,
                acc_sh, si0, si1, sg0, sg1):
    c = lax.axis_index("c")
    s = lax.axis_index("s")

    z = jnp.zeros((16,), _f32)

    def _zrow(r, _):
        for j in range(HD // 16):
            rows0[r, pl.ds(j * 16, 16)] = z
        return 0

    lax.fori_loop(0, K, _zrow, 0)
    nz = RPT // K
    rem = RPT - nz * K
    for r in range(nz):
        pltpu.async_copy(rows0, acc_sh.at[pl.ds(s * RPT + r * K, K)], si0)
    if rem:
        pltpu.async_copy(rows0.at[pl.ds(0, rem)],
                         acc_sh.at[pl.ds(s * RPT + nz * K, rem)], si0)
    for r in range(nz):
        pltpu.make_async_copy(
            rows0, acc_sh.at[pl.ds(s * RPT + r * K, K)], si0).wait()
    if rem:
        pltpu.make_async_copy(
            rows0.at[pl.ds(0, rem)],
            acc_sh.at[pl.ds(s * RPT + nz * K, rem)], si0).wait()
    plsc.subcore_barrier()

    sets = ((idx0, rows0, si0, sg0), (idx1, rows1, si1, sg1))

    def _start_idx(i, st):
        ev, _, si, _ = st
        base = s * EPT + i * K
        pltpu.async_copy(src_hbm.at[pl.ds(base, K)], ev.at[0], si)
        pltpu.async_copy(dst_hbm.at[pl.ds(base, K)], ev.at[1], si)
        pltpu.async_copy(w_hbm.at[pl.ds(base, K)], ev.at[2], si)

    def _wait_idx(st):
        ev, _, si, _ = st
        pltpu.make_async_copy(src_hbm.at[pl.ds(0, K)], ev.at[0], si).wait()
        pltpu.make_async_copy(dst_hbm.at[pl.ds(0, K)], ev.at[1], si).wait()
        pltpu.make_async_copy(w_hbm.at[pl.ds(0, K)], ev.at[2], si).wait()

    def _start_gather(st):
        ev, rv, _, sg = st
        pltpu.async_copy(h_hbm.at[c].at[ev.at[0]], rv, sg)

    def _wait_gather(st):
        ev, rv, _, sg = st
        pltpu.make_async_copy(h_hbm.at[c].at[ev.at[0]], rv, sg).wait()

    def _process(st):
        ev, rv, _, _ = st

        def _scale(g, _):
            wgrp = lax.bitcast_convert_type(ev[2, pl.ds(g * 16, 16)], _f32)
            for m in range(16):
                e = g * 16 + m
                we = wgrp[m]
                for j in range(HD // 16):
                    sl = pl.ds(j * 16, 16)
                    rv[e, sl] = rv[e, sl] * we
            return 0

        lax.fori_loop(0, K // 16, _scale, 0)
        pltpu.sync_copy(rv, acc_sh.at[ev.at[1]], add=True)

    _start_idx(0, sets[0])
    _wait_idx(sets[0])
    _start_gather(sets[0])
    _start_idx(1, sets[1])

    def _pair(p, _):
        _wait_idx(sets[1])
        _wait_gather(sets[0])
        _start_gather(sets[1])
        _process(sets[0])
        _start_idx(2 * p + 2, sets[0])
        _wait_idx(sets[0])
        _wait_gather(sets[1])
        _start_gather(sets[0])
        _process(sets[1])

        @pl.when(p < (NCHUNK - 1) // 2 - 1)
        def _():
            _start_idx(2 * p + 3, sets[1])

        return 0

    lax.fori_loop(0, (NCHUNK - 1) // 2, _pair, 0)
    _wait_gather(sets[0])
    _process(sets[0])
    plsc.subcore_barrier()

    sl = pl.ds(s * RPT, RPT)
    pltpu.sync_copy(acc_sh.at[sl], out_hbm.at[c].at[sl])


def _make_accum():
    mesh = plsc.VectorSubcoreMesh(core_axis_name="c", subcore_axis_name="s")
    return pl.kernel(
        _accum_body,
        out_type=jax.ShapeDtypeStruct((NC, NPAD, HD), _f32),
        mesh=mesh,
        compiler_params=pltpu.CompilerParams(use_tc_tiling_on_sc=False),
        scratch_types=[
            pltpu.VMEM((3, K), _i32),
            pltpu.VMEM((3, K), _i32),
            pltpu.VMEM((K, HD), _f32),
            pltpu.VMEM((K, HD), _f32),
            pltpu.VMEM_SHARED((NPAD, HD), _f32),
            pltpu.SemaphoreType.DMA,
            pltpu.SemaphoreType.DMA,
            pltpu.SemaphoreType.DMA,
            pltpu.SemaphoreType.DMA,
        ],
    )


def _norm_body(agg_ref, out_ref):
    x = agg_ref[...]
    ss = x[0] * x[0] + x[1] * x[1]
    g32 = jnp.arange(128) // HD
    bd = (g32[:, None] == g32[None, :]).astype(_f32)
    ss4 = jnp.dot(ss, bd, preferred_element_type=_f32)
    inv = 1.0 / jnp.maximum(jnp.sqrt(ss4), EPS)
    out_ref[...] = x * inv[None]


def _normalize(agg):
    aggr = agg.reshape(NC, NPAD // 4, 128)
    out = pl.pallas_call(
        _norm_body,
        grid=(NPAD // 4 // NROWBLK,),
        in_specs=[pl.BlockSpec((NC, NROWBLK, 128), lambda i: (0, i, 0))],
        out_specs=pl.BlockSpec((NC, NROWBLK, 128), lambda i: (0, i, 0)),
        out_shape=jax.ShapeDtypeStruct((NC, NPAD // 4, 128), _f32),
    )(aggr)
    return out.reshape(NC, NPAD, HD)


def _final_body(h0, h1, h2, h3, uid_hbm, pid_hbm, nid_hbm,
                pos_hbm, neg_hbm, reg_hbm,
                idx_v, ua_v, ub_v, pa_v, pb_v, na_v, nb_v,
                pos_v, neg_v, reg_v, sem):
    c = lax.axis_index("c")
    s = lax.axis_index("s")
    wid = s * NC + c
    base = wid * BW
    ii = _iota16()

    def _sumrows(id_hbm, bufa, bufb):
        pltpu.sync_copy(id_hbm.at[pl.ds(base, BW)], idx_v)
        pltpu.async_copy(h0.at[0].at[idx_v], bufa, sem).wait()
        pltpu.async_copy(h0.at[1].at[idx_v], bufb, sem).wait()
        for t in (h1, h2, h3):
            pltpu.async_copy(t.at[0].at[idx_v], bufa, sem, add=True).wait()
            pltpu.async_copy(t.at[1].at[idx_v], bufb, sem, add=True).wait()

    _sumrows(uid_hbm, ua_v, ub_v)
    _sumrows(pid_hbm, pa_v, pb_v)
    _sumrows(nid_hbm, na_v, nb_v)

    racc0 = jnp.zeros((16,), _f32)

    def _grp(g, racc):
        pos16 = jnp.zeros((16,), _f32)
        neg16 = jnp.zeros((16,), _f32)
        for m in range(16):
            e = g * 16 + m
            pacc = jnp.zeros((16,), _f32)
            nacc = jnp.zeros((16,), _f32)
            for (ub, pb, nb) in ((ua_v, pa_v, na_v), (ub_v, pb_v, nb_v)):
                for j in range(HD // 16):
                    sl = pl.ds(j * 16, 16)
                    gu = ub[e, sl]
                    gp = pb[e, sl]
                    gn = nb[e, sl]
                    pacc = pacc + gu * gp
                    nacc = nacc + gu * gn
                    racc = racc + gu * gu + gp * gp + gn * gn
            lane = ii == m
            pos16 = jnp.where(lane, _hsum16(pacc), pos16)
            neg16 = jnp.where(lane, _hsum16(nacc), neg16)
        pos_v[pl.ds(g * 16, 16)] = pos16 * 0.0625
        neg_v[pl.ds(g * 16, 16)] = neg16 * 0.0625
        return racc

    racc = lax.fori_loop(0, BW // 16, _grp, racc0)
    reg_v[pl.ds(0, 16)] = racc * 0.0625

    pltpu.sync_copy(pos_v, pos_hbm.at[pl.ds(base, BW)])
    pltpu.sync_copy(neg_v, neg_hbm.at[pl.ds(base, BW)])
    pltpu.sync_copy(reg_v, reg_hbm.at[wid])


def _make_final():
    mesh = plsc.VectorSubcoreMesh(core_axis_name="c", subcore_axis_name="s")
    return pl.kernel(
        _final_body,
        out_type=(
            jax.ShapeDtypeStruct((BATCH,), _f32),
            jax.ShapeDtypeStruct((BATCH,), _f32),
            jax.ShapeDtypeStruct((NC * NS, 16), _f32),
        ),
        mesh=mesh,
        compiler_params=pltpu.CompilerParams(use_tc_tiling_on_sc=False),
        scratch_types=[
            pltpu.VMEM((BW,), _i32),
            pltpu.VMEM((BW, HD), _f32),
            pltpu.VMEM((BW, HD), _f32),
            pltpu.VMEM((BW, HD), _f32),
            pltpu.VMEM((BW, HD), _f32),
            pltpu.VMEM((BW, HD), _f32),
            pltpu.VMEM((BW, HD), _f32),
            pltpu.VMEM((BW,), _f32),
            pltpu.VMEM((BW,), _f32),
            pltpu.VMEM((16,), _f32),
            pltpu.SemaphoreType.DMA,
        ],
    )


def _loss_body(pos_ref, neg_ref, reg_ref, out_ref):
    z = pos_ref[...] - neg_ref[...]
    loss = -jnp.mean(jnp.log(jax.nn.sigmoid(z)))
    reg = jnp.sum(reg_ref[...])
    out_ref[...] = jnp.full((1, 1), loss + LMBD * (reg * 0.5) / BATCH, _f32)


def kernel(edge_index, edge_weight, user_table, item_table,
           user_id, item_id, neg_item_id):
    src = edge_index[0].astype(_i32)
    dst = edge_index[1].astype(_i32)
    wbits = lax.bitcast_convert_type(edge_weight.astype(_f32), _i32)
    pad = jnp.zeros((NPAD - N_NODES, D), _f32)
    h0full = jnp.concatenate([user_table, item_table, pad], axis=0)
    h0 = jnp.stack([h0full[:, :HD], h0full[:, HD:]])

    accum = _make_accum()
    h1 = _normalize(accum(src, dst, wbits, h0))
    h2 = _normalize(accum(src, dst, wbits, h1))
    h3 = _normalize(accum(src, dst, wbits, h2))

    final = _make_final()
    pos, neg, regp = final(
        h0, h1, h2, h3,
        user_id.astype(_i32),
        (item_id + USER_N).astype(_i32),
        (neg_item_id + USER_N).astype(_i32),
    )

    out = pl.pallas_call(
        _loss_body,
        out_shape=jax.ShapeDtypeStruct((1, 1), _f32),
    )(pos.reshape(32, 128), neg.reshape(32, 128), regp)
    return out[0, 0]
